# baseline (speedup 1.0000x reference)
import dataclasses
import functools

import jax
import jax.numpy as jnp
from jax import lax
from jax.experimental import pallas as pl
from jax.experimental.pallas import tpu as pltpu
from jax.experimental.pallas import tpu_sc as plsc

_WD = 0.025
_NW = 32
_LANES = 16
_CHUNK = 128


def _sc_gather_dot(u, i, j, W, H, batch):
    rows_per_w = batch // _NW
    n_chunks = rows_per_w // _CHUNK
    dim = W.shape[1]
    mesh = plsc.VectorSubcoreMesh(core_axis_name="c", subcore_axis_name="s")
    cp = pltpu.CompilerParams()
    if "needs_layout_passes" in pltpu.CompilerParams.__dataclass_fields__:
        cp = dataclasses.replace(cp, needs_layout_passes=False)

    @functools.partial(
        pl.kernel,
        out_type=jax.ShapeDtypeStruct((batch * 2 * _LANES,), jnp.float32),
        mesh=mesh,
        compiler_params=cp,
        scratch_types=[
            pltpu.VMEM((rows_per_w,), jnp.int32),
            pltpu.VMEM((rows_per_w,), jnp.int32),
            pltpu.VMEM((rows_per_w,), jnp.int32),
            pltpu.VMEM((2, _CHUNK, dim), jnp.float32),
            pltpu.VMEM((2, _CHUNK, dim), jnp.float32),
            pltpu.VMEM((2, _CHUNK, dim), jnp.float32),
            pltpu.VMEM((rows_per_w * 2 * _LANES,), jnp.float32),
            pltpu.SemaphoreType.DMA((2,)),
            pltpu.SemaphoreType.DMA,
        ],
    )
    def sc_kernel(u_hbm, i_hbm, j_hbm, w_hbm, h_hbm, x_hbm,
                  uidx, iidx, jidx, urows, irows, jrows, xout,
                  gsem, isem):
        wid = lax.axis_index("s") * 2 + lax.axis_index("c")
        base = wid * rows_per_w
        ih = (
            pltpu.async_copy(u_hbm.at[pl.ds(base, rows_per_w)], uidx, isem),
            pltpu.async_copy(i_hbm.at[pl.ds(base, rows_per_w)], iidx, isem),
            pltpu.async_copy(j_hbm.at[pl.ds(base, rows_per_w)], jidx, isem),
        )
        for h in ih:
            h.wait()

        def fire_dyn(coff, b):
            pltpu.async_copy(
                w_hbm.at[uidx.at[pl.ds(coff, _CHUNK)]], urows.at[b],
                gsem.at[b])
            pltpu.async_copy(
                h_hbm.at[iidx.at[pl.ds(coff, _CHUNK)]], irows.at[b],
                gsem.at[b])
            pltpu.async_copy(
                h_hbm.at[jidx.at[pl.ds(coff, _CHUNK)]], jrows.at[b],
                gsem.at[b])

        def wait_gathers(b):
            pltpu.make_async_copy(
                w_hbm.at[uidx.at[pl.ds(0, _CHUNK)]], urows.at[b],
                gsem.at[b]).wait()
            pltpu.make_async_copy(
                h_hbm.at[iidx.at[pl.ds(0, _CHUNK)]], irows.at[b],
                gsem.at[b]).wait()
            pltpu.make_async_copy(
                h_hbm.at[jidx.at[pl.ds(0, _CHUNK)]], jrows.at[b],
                gsem.at[b]).wait()

        fire_dyn(0, 0)
        fire_dyn(_CHUNK, 1)

        @pl.loop(0, n_chunks)
        def _(c):
            k = lax.rem(c, 2)
            wait_gathers(k)

            @plsc.parallel_loop(0, _CHUNK, unroll=2)
            def _(r):
                acc = su = si = sj = None
                for d in range(dim // _LANES):
                    sl = pl.ds(d * _LANES, _LANES)
                    uv = urows[k, r, sl]
                    iv = irows[k, r, sl]
                    jv = jrows[k, r, sl]
                    px = uv * (iv - jv)
                    if d == 0:
                        acc, su, si, sj = px, uv * uv, iv * iv, jv * jv
                    else:
                        acc = acc + px
                        su = su + uv * uv
                        si = si + iv * iv
                        sj = sj + jv * jv
                ar = c * _CHUNK + r
                xout[pl.ds(ar * 2 * _LANES, _LANES)] = acc
                xout[pl.ds(ar * 2 * _LANES + _LANES, _LANES)] = (su + si) + sj

            @pl.when(c + 2 < n_chunks)
            def _():
                fire_dyn((c + 2) * _CHUNK, k)

        pltpu.sync_copy(
            xout, x_hbm.at[pl.ds(base * 2 * _LANES, rows_per_w * 2 * _LANES)])

    return sc_kernel(u, i, j, W, H)


def _tc_loss(xpart):
    def body(x_ref, o_ref):
        x2 = x_ref[...]
        lmap = lax.broadcasted_iota(jnp.int32, (128, 4), 0)
        gmap = lax.broadcasted_iota(jnp.int32, (128, 4), 1)
        sel = ((lmap // (2 * _LANES) == gmap)
               & (lmap % (2 * _LANES) < _LANES)).astype(jnp.bfloat16)
        logits = lax.dot_general(
            x2.astype(jnp.bfloat16), sel, (((1,), (0,)), ((), ())),
            preferred_element_type=jnp.float32)
        ls = jax.nn.log_sigmoid(logits)
        sum_logits = jnp.sum(logits)
        reg = _WD * (jnp.sum(x2) - sum_logits)
        o_ref[0, 0] = reg - jnp.sum(ls)

    out = pl.pallas_call(
        body,
        out_shape=jax.ShapeDtypeStruct((1, 1), jnp.float32),
        out_specs=pl.BlockSpec(memory_space=pltpu.SMEM),
    )(xpart.reshape(-1, 128))
    return out[0, 0]


def kernel(u, i, j, W, H):
    u = u.astype(jnp.int32)
    i = i.astype(jnp.int32)
    j = j.astype(jnp.int32)
    x = _sc_gather_dot(u, i, j, W, H, u.shape[0])
    return _tc_loss(x)


# speedup vs baseline: 1.0036x; 1.0036x over previous
import dataclasses
import functools

import jax
import jax.numpy as jnp
from jax import lax
from jax.experimental import pallas as pl
from jax.experimental.pallas import tpu as pltpu
from jax.experimental.pallas import tpu_sc as plsc

_WD = 0.025
_NW = 32
_LANES = 16
_CHUNK = 128


def _sc_gather_dot(u, i, j, W, H, batch):
    rows_per_w = batch // _NW
    n_chunks = rows_per_w // _CHUNK
    dim = W.shape[1]
    mesh = plsc.VectorSubcoreMesh(core_axis_name="c", subcore_axis_name="s")
    cp = pltpu.CompilerParams()
    if "needs_layout_passes" in pltpu.CompilerParams.__dataclass_fields__:
        cp = dataclasses.replace(cp, needs_layout_passes=False)

    @functools.partial(
        pl.kernel,
        out_type=jax.ShapeDtypeStruct((batch * 2 * _LANES,), jnp.float32),
        mesh=mesh,
        compiler_params=cp,
        scratch_types=[
            pltpu.VMEM((rows_per_w,), jnp.int32),
            pltpu.VMEM((rows_per_w,), jnp.int32),
            pltpu.VMEM((rows_per_w,), jnp.int32),
            pltpu.VMEM((2, _CHUNK, dim), jnp.float32),
            pltpu.VMEM((2, _CHUNK, dim), jnp.float32),
            pltpu.VMEM((2, _CHUNK, dim), jnp.float32),
            pltpu.VMEM((rows_per_w * 2 * _LANES,), jnp.float32),
            pltpu.SemaphoreType.DMA((2,)),
            pltpu.SemaphoreType.DMA,
        ],
    )
    def sc_kernel(u_hbm, i_hbm, j_hbm, w_hbm, h_hbm, x_hbm,
                  uidx, iidx, jidx, urows, irows, jrows, xout,
                  gsem, isem):
        wid = lax.axis_index("s") * 2 + lax.axis_index("c")
        base = wid * rows_per_w
        ih = (
            pltpu.async_copy(u_hbm.at[pl.ds(base, rows_per_w)], uidx, isem),
            pltpu.async_copy(i_hbm.at[pl.ds(base, rows_per_w)], iidx, isem),
            pltpu.async_copy(j_hbm.at[pl.ds(base, rows_per_w)], jidx, isem),
        )
        for h in ih:
            h.wait()

        def fire_dyn(coff, b):
            pltpu.async_copy(
                w_hbm.at[uidx.at[pl.ds(coff, _CHUNK)]], urows.at[b],
                gsem.at[b])
            pltpu.async_copy(
                h_hbm.at[iidx.at[pl.ds(coff, _CHUNK)]], irows.at[b],
                gsem.at[b])
            pltpu.async_copy(
                h_hbm.at[jidx.at[pl.ds(coff, _CHUNK)]], jrows.at[b],
                gsem.at[b])

        def wait_gathers(b):
            pltpu.make_async_copy(
                w_hbm.at[uidx.at[pl.ds(0, _CHUNK)]], urows.at[b],
                gsem.at[b]).wait()
            pltpu.make_async_copy(
                h_hbm.at[iidx.at[pl.ds(0, _CHUNK)]], irows.at[b],
                gsem.at[b]).wait()
            pltpu.make_async_copy(
                h_hbm.at[jidx.at[pl.ds(0, _CHUNK)]], jrows.at[b],
                gsem.at[b]).wait()

        fire_dyn(0, 0)
        fire_dyn(_CHUNK, 1)

        @pl.loop(0, n_chunks)
        def _(c):
            k = lax.rem(c, 2)
            wait_gathers(k)

            @plsc.parallel_loop(0, _CHUNK, unroll=1)
            def _(r):
                acc = su = si = sj = None
                for d in range(dim // _LANES):
                    sl = pl.ds(d * _LANES, _LANES)
                    uv = urows[k, r, sl]
                    iv = irows[k, r, sl]
                    jv = jrows[k, r, sl]
                    px = uv * (iv - jv)
                    if d == 0:
                        acc, su, si, sj = px, uv * uv, iv * iv, jv * jv
                    else:
                        acc = acc + px
                        su = su + uv * uv
                        si = si + iv * iv
                        sj = sj + jv * jv
                ar = c * _CHUNK + r
                xout[pl.ds(ar * 2 * _LANES, _LANES)] = acc
                xout[pl.ds(ar * 2 * _LANES + _LANES, _LANES)] = (su + si) + sj

            @pl.when(c + 2 < n_chunks)
            def _():
                fire_dyn((c + 2) * _CHUNK, k)

        pltpu.sync_copy(
            xout, x_hbm.at[pl.ds(base * 2 * _LANES, rows_per_w * 2 * _LANES)])

    return sc_kernel(u, i, j, W, H)


def _tc_loss(xpart):
    def body(x_ref, o_ref):
        x2 = x_ref[...]
        lmap = lax.broadcasted_iota(jnp.int32, (128, 4), 0)
        gmap = lax.broadcasted_iota(jnp.int32, (128, 4), 1)
        sel = ((lmap // (2 * _LANES) == gmap)
               & (lmap % (2 * _LANES) < _LANES)).astype(jnp.bfloat16)
        logits = lax.dot_general(
            x2.astype(jnp.bfloat16), sel, (((1,), (0,)), ((), ())),
            preferred_element_type=jnp.float32)
        ls = jax.nn.log_sigmoid(logits)
        sum_logits = jnp.sum(logits)
        reg = _WD * (jnp.sum(x2) - sum_logits)
        o_ref[0, 0] = reg - jnp.sum(ls)

    out = pl.pallas_call(
        body,
        out_shape=jax.ShapeDtypeStruct((1, 1), jnp.float32),
        out_specs=pl.BlockSpec(memory_space=pltpu.SMEM),
    )(xpart.reshape(-1, 128))
    return out[0, 0]


def kernel(u, i, j, W, H):
    u = u.astype(jnp.int32)
    i = i.astype(jnp.int32)
    j = j.astype(jnp.int32)
    x = _sc_gather_dot(u, i, j, W, H, u.shape[0])
    return _tc_loss(x)


# speedup vs baseline: 1.0082x; 1.0046x over previous
import dataclasses
import functools

import jax
import jax.numpy as jnp
from jax import lax
from jax.experimental import pallas as pl
from jax.experimental.pallas import tpu as pltpu
from jax.experimental.pallas import tpu_sc as plsc

_WD = 0.025
_NW = 32
_LANES = 16
_CHUNK = 128


def _sc_gather_dot(u, i, j, W, H, batch):
    rows_per_w = batch // _NW
    n_chunks = rows_per_w // _CHUNK
    dim = W.shape[1]
    mesh = plsc.VectorSubcoreMesh(core_axis_name="c", subcore_axis_name="s")
    cp = pltpu.CompilerParams()
    if "needs_layout_passes" in pltpu.CompilerParams.__dataclass_fields__:
        cp = dataclasses.replace(cp, needs_layout_passes=False)

    @functools.partial(
        pl.kernel,
        out_type=jax.ShapeDtypeStruct((batch * 2 * _LANES,), jnp.float32),
        mesh=mesh,
        compiler_params=cp,
        scratch_types=[
            pltpu.VMEM((rows_per_w,), jnp.int32),
            pltpu.VMEM((rows_per_w,), jnp.int32),
            pltpu.VMEM((rows_per_w,), jnp.int32),
            pltpu.VMEM((2, _CHUNK, dim), jnp.float32),
            pltpu.VMEM((2, _CHUNK, dim), jnp.float32),
            pltpu.VMEM((2, _CHUNK, dim), jnp.float32),
            pltpu.VMEM((rows_per_w * 2 * _LANES,), jnp.float32),
            pltpu.SemaphoreType.DMA((2,)),
            pltpu.SemaphoreType.DMA,
        ],
    )
    def sc_kernel(u_hbm, i_hbm, j_hbm, w_hbm, h_hbm, x_hbm,
                  uidx, iidx, jidx, urows, irows, jrows, xout,
                  gsem, isem):
        wid = lax.axis_index("s") * 2 + lax.axis_index("c")
        base = wid * rows_per_w
        ih = (
            pltpu.async_copy(u_hbm.at[pl.ds(base, rows_per_w)], uidx, isem),
            pltpu.async_copy(i_hbm.at[pl.ds(base, rows_per_w)], iidx, isem),
            pltpu.async_copy(j_hbm.at[pl.ds(base, rows_per_w)], jidx, isem),
        )
        for h in ih:
            h.wait()

        def fire_dyn(coff, b):
            pltpu.async_copy(
                w_hbm.at[uidx.at[pl.ds(coff, _CHUNK)]], urows.at[b],
                gsem.at[b])
            pltpu.async_copy(
                h_hbm.at[iidx.at[pl.ds(coff, _CHUNK)]], irows.at[b],
                gsem.at[b])
            pltpu.async_copy(
                h_hbm.at[jidx.at[pl.ds(coff, _CHUNK)]], jrows.at[b],
                gsem.at[b])

        def wait_gathers(b):
            pltpu.make_async_copy(
                w_hbm.at[uidx.at[pl.ds(0, _CHUNK)]], urows.at[b],
                gsem.at[b]).wait()
            pltpu.make_async_copy(
                h_hbm.at[iidx.at[pl.ds(0, _CHUNK)]], irows.at[b],
                gsem.at[b]).wait()
            pltpu.make_async_copy(
                h_hbm.at[jidx.at[pl.ds(0, _CHUNK)]], jrows.at[b],
                gsem.at[b]).wait()

        fire_dyn(0, 0)
        fire_dyn(_CHUNK, 1)

        @pl.loop(0, n_chunks)
        def _(c):
            k = lax.rem(c, 2)
            wait_gathers(k)

            @plsc.parallel_loop(0, _CHUNK, unroll=1)
            def _(r):
                acc = su = si = sj = None
                for d in range(dim // _LANES):
                    sl = pl.ds(d * _LANES, _LANES)
                    uv = urows[k, r, sl]
                    iv = irows[k, r, sl]
                    jv = jrows[k, r, sl]
                    px = uv * (iv - jv)
                    if d == 0:
                        acc, su, si, sj = px, uv * uv, iv * iv, jv * jv
                    else:
                        acc = acc + px
                        su = su + uv * uv
                        si = si + iv * iv
                        sj = sj + jv * jv
                ar = c * _CHUNK + r
                xout[pl.ds(ar * 2 * _LANES, _LANES)] = acc
                xout[pl.ds(ar * 2 * _LANES + _LANES, _LANES)] = (su + si) + sj

            @pl.when(c + 2 < n_chunks)
            def _():
                fire_dyn((c + 2) * _CHUNK, k)

        pltpu.sync_copy(
            xout, x_hbm.at[pl.ds(base * 2 * _LANES, rows_per_w * 2 * _LANES)])

    return sc_kernel(u, i, j, W, H)


def _tc_loss(xpart):
    def body(x_ref, o_ref):
        x2 = x_ref[...]
        lmap = lax.broadcasted_iota(jnp.int32, (128, 8), 0)
        gmap = lax.broadcasted_iota(jnp.int32, (128, 8), 1)
        sel = ((lmap // (2 * _LANES) == gmap % 4)
               & ((lmap % (2 * _LANES) < _LANES) == (gmap < 4))
               ).astype(jnp.bfloat16)
        both = lax.dot_general(
            x2.astype(jnp.bfloat16), sel, (((1,), (0,)), ((), ())),
            preferred_element_type=jnp.float32)
        ls = jax.nn.log_sigmoid(both[:, :4])
        reg = _WD * jnp.sum(both[:, 4:])
        o_ref[0, 0] = reg - jnp.sum(ls)

    out = pl.pallas_call(
        body,
        out_shape=jax.ShapeDtypeStruct((1, 1), jnp.float32),
        out_specs=pl.BlockSpec(memory_space=pltpu.SMEM),
    )(xpart.reshape(-1, 128))
    return out[0, 0]


def kernel(u, i, j, W, H):
    u = u.astype(jnp.int32)
    i = i.astype(jnp.int32)
    j = j.astype(jnp.int32)
    x = _sc_gather_dot(u, i, j, W, H, u.shape[0])
    return _tc_loss(x)
